# Initial kernel scaffold; baseline (speedup 1.0000x reference)
import functools

import jax
import jax.numpy as jnp
from jax import lax
from jax.experimental import pallas as pl
from jax.experimental.pallas import tpu as pltpu
from jax.experimental.pallas import tpu_sc as plsc


@functools.lru_cache(maxsize=None)
def _edge_proj(E, DE, D, BE):
    F = 128 // DE
    R = E // F
    G = R // BE
    assert R % BE == 0

    def kern(ea_ref, w_ref, b_ref, o_ref):
        o_ref[...] = (
            jnp.dot(ea_ref[...], w_ref[...], preferred_element_type=jnp.float32)
            + b_ref[...]
        )

    return pl.pallas_call(
        kern,
        grid=(G,),
        in_specs=[
            pl.BlockSpec((BE, F * DE), lambda i: (i, 0)),
            pl.BlockSpec((F * DE, F * D), lambda i: (0, 0)),
            pl.BlockSpec((1, F * D), lambda i: (0, 0)),
        ],
        out_specs=pl.BlockSpec((BE, F * D), lambda i: (i, 0)),
        out_shape=jax.ShapeDtypeStruct((R, F * D), jnp.float32),
    )


@functools.lru_cache(maxsize=None)
def _sc_edge_agg(N, D, E):
    info = plsc.get_sparse_core_info()
    NC, NS = info.num_cores, info.num_subcores
    NW = NC * NS
    EPW = E // NW
    CH = 80
    NCHUNK = EPW // CH
    assert EPW * NW == E and NCHUNK * CH == EPW
    RPT = N // NS
    assert RPT * NS == N
    ZR = 125
    assert RPT % ZR == 0
    NZ = RPT // ZR
    NV = D // 16

    mesh = plsc.VectorSubcoreMesh(core_axis_name="c", subcore_axis_name="s")

    @functools.partial(
        pl.kernel,
        mesh=mesh,
        out_type=jax.ShapeDtypeStruct((NC, N, D), jnp.float32),
        scratch_types=[
            pltpu.VMEM_SHARED((N, D), jnp.float32),
            pltpu.VMEM((CH,), jnp.int32),
            pltpu.VMEM((CH,), jnp.int32),
            pltpu.VMEM((CH, D), jnp.float32),
            pltpu.VMEM((CH, D), jnp.float32),
            pltpu.VMEM((ZR, D), jnp.float32),
        ],
    )
    def sc_edge(x_hbm, src_hbm, dst_hbm, e_hbm, out_hbm,
                agg, srcb, dstb, xgb, eb, zb):
        cid = lax.axis_index("c")
        sid = lax.axis_index("s")
        wid = sid * NC + cid

        def zrow(r, carry):
            for c in range(NV):
                zb[r, pl.ds(c * 16, 16)] = jnp.zeros((16,), jnp.float32)
            return carry

        lax.fori_loop(0, ZR, zrow, 0)
        for j in range(NZ):
            pltpu.sync_copy(zb, agg.at[pl.ds(sid * RPT + j * ZR, ZR)])
        plsc.subcore_barrier()

        ebase = wid * EPW

        def body(i, carry):
            base = ebase + i * CH
            pltpu.sync_copy(src_hbm.at[pl.ds(base, CH)], srcb)
            pltpu.sync_copy(dst_hbm.at[pl.ds(base, CH)], dstb)
            pltpu.sync_copy(e_hbm.at[pl.ds(base, CH)], eb)
            pltpu.sync_copy(x_hbm.at[srcb], xgb)

            def crow(r, c2):
                for c in range(NV):
                    sl = pl.ds(c * 16, 16)
                    xgb[r, sl] = jnp.maximum(xgb[r, sl] + eb[r, sl], 0.0)
                return c2

            lax.fori_loop(0, CH, crow, 0)
            pltpu.sync_copy(xgb, agg.at[dstb], add=True)
            return carry

        lax.fori_loop(0, NCHUNK, body, 0)

        plsc.subcore_barrier()
        for j in range(NZ):
            sl = pl.ds(sid * RPT + j * ZR, ZR)
            pltpu.sync_copy(agg.at[sl], out_hbm.at[cid, sl])

    return sc_edge


@functools.lru_cache(maxsize=None)
def _dense1(N, D, B, BN):
    G = N // BN
    SW = 3 * D

    def kern(x_ref, agg_ref, bt_ref, w1_ref, b1_ref, w2_ref, b2_ref,
             h2_ref, st_ref):
        h = x_ref[...] + agg_ref[0] + agg_ref[1]
        t = jnp.dot(h, w1_ref[...], preferred_element_type=jnp.float32) + b1_ref[...]
        t = t * jax.nn.sigmoid(t)
        h2 = jnp.dot(t, w2_ref[...], preferred_element_type=jnp.float32) + b2_ref[...]
        h2_ref[...] = h2
        bt = bt_ref[0, 0, :]
        oh = (lax.broadcasted_iota(jnp.int32, (B, BN), 0) == bt[None, :]).astype(
            jnp.float32)
        cat = jnp.concatenate([h2, h2 * h2, jnp.ones((BN, D), jnp.float32)], axis=1)

        @pl.when(pl.program_id(0) == 0)
        def _init():
            st_ref[...] = jnp.zeros_like(st_ref)

        st_ref[...] += jnp.dot(oh, cat, preferred_element_type=jnp.float32)

    return pl.pallas_call(
        kern,
        grid=(G,),
        in_specs=[
            pl.BlockSpec((BN, D), lambda i: (i, 0)),
            pl.BlockSpec((2, BN, D), lambda i: (0, i, 0)),
            pl.BlockSpec((1, 1, BN), lambda i: (i, 0, 0)),
            pl.BlockSpec((D, D), lambda i: (0, 0)),
            pl.BlockSpec((1, D), lambda i: (0, 0)),
            pl.BlockSpec((D, D), lambda i: (0, 0)),
            pl.BlockSpec((1, D), lambda i: (0, 0)),
        ],
        out_specs=[
            pl.BlockSpec((BN, D), lambda i: (i, 0)),
            pl.BlockSpec((B, SW), lambda i: (0, 0)),
        ],
        out_shape=[
            jax.ShapeDtypeStruct((N, D), jnp.float32),
            jax.ShapeDtypeStruct((B, SW), jnp.float32),
        ],
    )


@functools.lru_cache(maxsize=None)
def _dense2(N, D, B, TD, BN):
    G = N // BN
    SW = 3 * D

    def kern(x_ref, h2_ref, bt_ref, st_ref, te_ref, gw_ref, gb_ref,
             bw_ref, bb_ref, gnw_ref, gnb_ref, gms_ref, o_ref, ac_ref):
        @pl.when(pl.program_id(0) == 0)
        def _coef():
            counts = jnp.maximum(st_ref[:, 2 * D:3 * D][:, 0:1], 1.0)
            m = st_ref[:, 0:D] / counts
            eh2 = st_ref[:, D:2 * D] / counts
            ms = gms_ref[...]
            var = eh2 - m * m * ms * (2.0 - ms)
            rstd = lax.rsqrt(var + 1e-5)
            gamma = (jnp.dot(te_ref[...], gw_ref[...],
                             preferred_element_type=jnp.float32)
                     + gb_ref[...] + 1.0)
            beta = (jnp.dot(te_ref[...], bw_ref[...],
                            preferred_element_type=jnp.float32)
                    + bb_ref[...])
            w = gnw_ref[...]
            a = gamma * rstd * w
            cc = gamma * (gnb_ref[...] - m * ms * rstd * w) + beta
            ac_ref[...] = jnp.concatenate([a, cc], axis=1)

        bt = bt_ref[0, 0, :]
        oh = (lax.broadcasted_iota(jnp.int32, (BN, B), 1) == bt[:, None]).astype(
            jnp.float32)
        acg = jnp.dot(oh, ac_ref[...], preferred_element_type=jnp.float32)
        z = acg[:, 0:D] * h2_ref[...] + acg[:, D:2 * D]
        o_ref[...] = x_ref[...] + z * jax.nn.sigmoid(z)

    return pl.pallas_call(
        kern,
        grid=(G,),
        in_specs=[
            pl.BlockSpec((BN, D), lambda i: (i, 0)),
            pl.BlockSpec((BN, D), lambda i: (i, 0)),
            pl.BlockSpec((1, 1, BN), lambda i: (i, 0, 0)),
            pl.BlockSpec((B, SW), lambda i: (0, 0)),
            pl.BlockSpec((B, TD), lambda i: (0, 0)),
            pl.BlockSpec((TD, D), lambda i: (0, 0)),
            pl.BlockSpec((1, D), lambda i: (0, 0)),
            pl.BlockSpec((TD, D), lambda i: (0, 0)),
            pl.BlockSpec((1, D), lambda i: (0, 0)),
            pl.BlockSpec((1, D), lambda i: (0, 0)),
            pl.BlockSpec((1, D), lambda i: (0, 0)),
            pl.BlockSpec((1, D), lambda i: (0, 0)),
        ],
        out_specs=pl.BlockSpec((BN, D), lambda i: (i, 0)),
        out_shape=jax.ShapeDtypeStruct((N, D), jnp.float32),
        scratch_shapes=[pltpu.VMEM((B, 2 * D), jnp.float32)],
    )


def kernel(x, edge_index, edge_attr, batch, target_embeddings,
           lin_edge_W, lin_edge_b, nn_W1, nn_b1, nn_W2, nn_b2,
           gn_weight, gn_bias, gn_mean_scale,
           film_gamma_W, film_gamma_b, film_beta_W, film_beta_b):
    N, D = x.shape
    E = edge_index.shape[1]
    DE = edge_attr.shape[1]
    B, TD = target_embeddings.shape
    F = 128 // DE

    src = edge_index[0]
    dst = edge_index[1]

    w_big = jnp.kron(jnp.eye(F, dtype=jnp.float32), lin_edge_W)
    b_big = jnp.tile(lin_edge_b, F).reshape(1, F * D)
    ea2 = edge_attr.reshape(E // F, F * DE)
    e = _edge_proj(E, DE, D, 500)(ea2, w_big, b_big).reshape(E, D)

    agg_p = _sc_edge_agg(N, D, E)(x, src, dst, e)

    BN = 1000
    batch3 = batch.reshape(N // BN, 1, BN)
    h2, stats = _dense1(N, D, B, BN)(
        x, agg_p, batch3, nn_W1, nn_b1.reshape(1, D), nn_W2, nn_b2.reshape(1, D))
    out = _dense2(N, D, B, TD, BN)(
        x, h2, batch3, stats, target_embeddings,
        film_gamma_W, film_gamma_b.reshape(1, D),
        film_beta_W, film_beta_b.reshape(1, D),
        gn_weight.reshape(1, D), gn_bias.reshape(1, D), gn_mean_scale.reshape(1, D))
    return out


# speedup vs baseline: 2.2076x; 2.2076x over previous
import functools

import jax
import jax.numpy as jnp
from jax import lax
from jax.experimental import pallas as pl
from jax.experimental.pallas import tpu as pltpu
from jax.experimental.pallas import tpu_sc as plsc


@functools.lru_cache(maxsize=None)
def _edge_proj(E, DE, D, BE):
    F = 128 // DE
    R = E // F
    G = R // BE
    assert R % BE == 0

    def kern(ea_ref, w_ref, b_ref, o_ref):
        o_ref[...] = (
            jnp.dot(ea_ref[...], w_ref[...], preferred_element_type=jnp.float32)
            + b_ref[...]
        )

    return pl.pallas_call(
        kern,
        grid=(G,),
        in_specs=[
            pl.BlockSpec((BE, F * DE), lambda i: (i, 0)),
            pl.BlockSpec((F * DE, F * D), lambda i: (0, 0)),
            pl.BlockSpec((1, F * D), lambda i: (0, 0)),
        ],
        out_specs=pl.BlockSpec((BE, F * D), lambda i: (i, 0)),
        out_shape=jax.ShapeDtypeStruct((R, F * D), jnp.float32),
    )


@functools.lru_cache(maxsize=None)
def _sc_edge_agg(N, D, E):
    info = plsc.get_sparse_core_info()
    NC, NS = info.num_cores, info.num_subcores
    NW = NC * NS
    EPW = E // NW
    CH = 80
    NCHUNK = EPW // CH
    assert EPW * NW == E and NCHUNK * CH == EPW
    ZR = 80
    NZCH = N // ZR
    assert NZCH * ZR == N
    NZ = -(-NZCH // NS)
    NV = D // 16

    mesh = plsc.VectorSubcoreMesh(core_axis_name="c", subcore_axis_name="s")

    @functools.partial(
        pl.kernel,
        mesh=mesh,
        out_type=jax.ShapeDtypeStruct((NC, N, D), jnp.float32),
        scratch_types=[
            pltpu.VMEM_SHARED((N, D), jnp.float32),
            pltpu.VMEM((CH,), jnp.int32),
            pltpu.VMEM((CH,), jnp.int32),
            pltpu.VMEM((CH, D), jnp.float32),
            pltpu.VMEM((CH, D), jnp.float32),
            pltpu.VMEM((ZR, D), jnp.float32),
        ],
    )
    def sc_edge(x_hbm, src_hbm, dst_hbm, e_hbm, out_hbm,
                agg, srcb, dstb, xgb, eb, zb):
        cid = lax.axis_index("c")
        sid = lax.axis_index("s")
        wid = sid * NC + cid

        def zrow(r, carry):
            for c in range(NV):
                zb[r, pl.ds(c * 16, 16)] = jnp.zeros((16,), jnp.float32)
            return carry

        lax.fori_loop(0, ZR, zrow, 0)
        for j in range(NZ):
            ch = sid + j * NS

            @pl.when(ch < NZCH)
            def _z():
                pltpu.sync_copy(zb, agg.at[pl.ds(ch * ZR, ZR)])

        plsc.subcore_barrier()

        ebase = wid * EPW

        def body(i, carry):
            base = ebase + i * CH
            pltpu.sync_copy(src_hbm.at[pl.ds(base, CH)], srcb)
            pltpu.sync_copy(dst_hbm.at[pl.ds(base, CH)], dstb)
            pltpu.sync_copy(e_hbm.at[pl.ds(base, CH)], eb)
            pltpu.sync_copy(x_hbm.at[srcb], xgb)

            def crow(r, c2):
                for c in range(NV):
                    sl = pl.ds(c * 16, 16)
                    xgb[r, sl] = jnp.maximum(xgb[r, sl] + eb[r, sl], 0.0)
                return c2

            lax.fori_loop(0, CH, crow, 0)
            pltpu.sync_copy(xgb, agg.at[dstb], add=True)
            return carry

        lax.fori_loop(0, NCHUNK, body, 0)

        plsc.subcore_barrier()
        for j in range(NZ):
            ch = sid + j * NS

            @pl.when(ch < NZCH)
            def _o():
                sl = pl.ds(ch * ZR, ZR)
                pltpu.sync_copy(agg.at[sl], out_hbm.at[cid, sl])

    return sc_edge


@functools.lru_cache(maxsize=None)
def _dense1(N, D, B, BN):
    G = N // BN
    SW = 3 * D

    def kern(x_ref, agg_ref, bt_ref, w1_ref, b1_ref, w2_ref, b2_ref,
             h2_ref, st_ref):
        h = x_ref[...] + agg_ref[0] + agg_ref[1]
        t = jnp.dot(h, w1_ref[...], preferred_element_type=jnp.float32) + b1_ref[...]
        t = t * jax.nn.sigmoid(t)
        h2 = jnp.dot(t, w2_ref[...], preferred_element_type=jnp.float32) + b2_ref[...]
        h2_ref[...] = h2
        bt = bt_ref[0, 0, :]
        oh = (lax.broadcasted_iota(jnp.int32, (B, BN), 0) == bt[None, :]).astype(
            jnp.float32)
        cat = jnp.concatenate([h2, h2 * h2, jnp.ones((BN, D), jnp.float32)], axis=1)

        @pl.when(pl.program_id(0) == 0)
        def _init():
            st_ref[...] = jnp.zeros_like(st_ref)

        st_ref[...] += jnp.dot(oh, cat, preferred_element_type=jnp.float32)

    return pl.pallas_call(
        kern,
        grid=(G,),
        in_specs=[
            pl.BlockSpec((BN, D), lambda i: (i, 0)),
            pl.BlockSpec((2, BN, D), lambda i: (0, i, 0)),
            pl.BlockSpec((1, 1, BN), lambda i: (i, 0, 0)),
            pl.BlockSpec((D, D), lambda i: (0, 0)),
            pl.BlockSpec((1, D), lambda i: (0, 0)),
            pl.BlockSpec((D, D), lambda i: (0, 0)),
            pl.BlockSpec((1, D), lambda i: (0, 0)),
        ],
        out_specs=[
            pl.BlockSpec((BN, D), lambda i: (i, 0)),
            pl.BlockSpec((B, SW), lambda i: (0, 0)),
        ],
        out_shape=[
            jax.ShapeDtypeStruct((N, D), jnp.float32),
            jax.ShapeDtypeStruct((B, SW), jnp.float32),
        ],
    )


@functools.lru_cache(maxsize=None)
def _dense2(N, D, B, TD, BN):
    G = N // BN
    SW = 3 * D

    def kern(x_ref, h2_ref, bt_ref, st_ref, te_ref, gw_ref, gb_ref,
             bw_ref, bb_ref, gnw_ref, gnb_ref, gms_ref, o_ref, ac_ref):
        @pl.when(pl.program_id(0) == 0)
        def _coef():
            counts = jnp.maximum(st_ref[:, 2 * D:3 * D][:, 0:1], 1.0)
            m = st_ref[:, 0:D] / counts
            eh2 = st_ref[:, D:2 * D] / counts
            ms = gms_ref[...]
            var = eh2 - m * m * ms * (2.0 - ms)
            rstd = lax.rsqrt(var + 1e-5)
            gamma = (jnp.dot(te_ref[...], gw_ref[...],
                             preferred_element_type=jnp.float32)
                     + gb_ref[...] + 1.0)
            beta = (jnp.dot(te_ref[...], bw_ref[...],
                            preferred_element_type=jnp.float32)
                    + bb_ref[...])
            w = gnw_ref[...]
            a = gamma * rstd * w
            cc = gamma * (gnb_ref[...] - m * ms * rstd * w) + beta
            ac_ref[...] = jnp.concatenate([a, cc], axis=1)

        bt = bt_ref[0, 0, :]
        oh = (lax.broadcasted_iota(jnp.int32, (BN, B), 1) == bt[:, None]).astype(
            jnp.float32)
        acg = jnp.dot(oh, ac_ref[...], preferred_element_type=jnp.float32)
        z = acg[:, 0:D] * h2_ref[...] + acg[:, D:2 * D]
        o_ref[...] = x_ref[...] + z * jax.nn.sigmoid(z)

    return pl.pallas_call(
        kern,
        grid=(G,),
        in_specs=[
            pl.BlockSpec((BN, D), lambda i: (i, 0)),
            pl.BlockSpec((BN, D), lambda i: (i, 0)),
            pl.BlockSpec((1, 1, BN), lambda i: (i, 0, 0)),
            pl.BlockSpec((B, SW), lambda i: (0, 0)),
            pl.BlockSpec((B, TD), lambda i: (0, 0)),
            pl.BlockSpec((TD, D), lambda i: (0, 0)),
            pl.BlockSpec((1, D), lambda i: (0, 0)),
            pl.BlockSpec((TD, D), lambda i: (0, 0)),
            pl.BlockSpec((1, D), lambda i: (0, 0)),
            pl.BlockSpec((1, D), lambda i: (0, 0)),
            pl.BlockSpec((1, D), lambda i: (0, 0)),
            pl.BlockSpec((1, D), lambda i: (0, 0)),
        ],
        out_specs=pl.BlockSpec((BN, D), lambda i: (i, 0)),
        out_shape=jax.ShapeDtypeStruct((N, D), jnp.float32),
        scratch_shapes=[pltpu.VMEM((B, 2 * D), jnp.float32)],
    )


def kernel(x, edge_index, edge_attr, batch, target_embeddings,
           lin_edge_W, lin_edge_b, nn_W1, nn_b1, nn_W2, nn_b2,
           gn_weight, gn_bias, gn_mean_scale,
           film_gamma_W, film_gamma_b, film_beta_W, film_beta_b):
    N, D = x.shape
    E = edge_index.shape[1]
    DE = edge_attr.shape[1]
    B, TD = target_embeddings.shape
    F = 128 // DE

    src = edge_index[0]
    dst = edge_index[1]

    w_big = jnp.kron(jnp.eye(F, dtype=jnp.float32), lin_edge_W)
    b_big = jnp.tile(lin_edge_b, F).reshape(1, F * D)
    ea2 = edge_attr.reshape(E // F, F * DE)
    e = _edge_proj(E, DE, D, 400)(ea2, w_big, b_big).reshape(E, D)

    agg_p = _sc_edge_agg(N, D, E)(x, src, dst, e)

    BN = 1000
    batch3 = batch.reshape(N // BN, 1, BN)
    h2, stats = _dense1(N, D, B, BN)(
        x, agg_p, batch3, nn_W1, nn_b1.reshape(1, D), nn_W2, nn_b2.reshape(1, D))
    out = _dense2(N, D, B, TD, BN)(
        x, h2, batch3, stats, target_embeddings,
        film_gamma_W, film_gamma_b.reshape(1, D),
        film_beta_W, film_beta_b.reshape(1, D),
        gn_weight.reshape(1, D), gn_bias.reshape(1, D), gn_mean_scale.reshape(1, D))
    return out


# speedup vs baseline: 3.0051x; 1.3613x over previous
import functools

import jax
import jax.numpy as jnp
from jax import lax
from jax.experimental import pallas as pl
from jax.experimental.pallas import tpu as pltpu
from jax.experimental.pallas import tpu_sc as plsc


@functools.lru_cache(maxsize=None)
def _edge_proj(E, DE, D, BE):
    F = 128 // DE
    R = E // F
    G = R // BE
    assert R % BE == 0

    def kern(ea_ref, w_ref, b_ref, o_ref):
        o_ref[...] = (
            jnp.dot(ea_ref[...], w_ref[...], preferred_element_type=jnp.float32)
            + b_ref[...]
        )

    return pl.pallas_call(
        kern,
        grid=(G,),
        in_specs=[
            pl.BlockSpec((BE, F * DE), lambda i: (i, 0)),
            pl.BlockSpec((F * DE, F * D), lambda i: (0, 0)),
            pl.BlockSpec((1, F * D), lambda i: (0, 0)),
        ],
        out_specs=pl.BlockSpec((BE, F * D), lambda i: (i, 0)),
        out_shape=jax.ShapeDtypeStruct((R, F * D), jnp.float32),
    )


@functools.lru_cache(maxsize=None)
def _sc_edge_agg(N, D, E):
    info = plsc.get_sparse_core_info()
    NC, NS = info.num_cores, info.num_subcores
    NW = NC * NS
    EPW = E // NW
    CH = 80
    NCHUNK = EPW // CH
    assert EPW * NW == E and NCHUNK * CH == EPW
    ZR = CH
    NZCH = N // ZR
    assert NZCH * ZR == N
    NZ = -(-NZCH // NS)
    NV = D // 16

    mesh = plsc.VectorSubcoreMesh(core_axis_name="c", subcore_axis_name="s")

    @functools.partial(
        pl.kernel,
        mesh=mesh,
        out_type=jax.ShapeDtypeStruct((NC, N, D), jnp.float32),
        scratch_types=[
            pltpu.VMEM_SHARED((N, D), jnp.float32),
            pltpu.VMEM((CH,), jnp.int32),
            pltpu.VMEM((CH,), jnp.int32),
            pltpu.VMEM((CH,), jnp.int32),
            pltpu.VMEM((CH,), jnp.int32),
            pltpu.VMEM((CH, D), jnp.float32),
            pltpu.VMEM((CH, D), jnp.float32),
            pltpu.VMEM((CH, D), jnp.float32),
            pltpu.VMEM((CH, D), jnp.float32),
            pltpu.SemaphoreType.DMA,
            pltpu.SemaphoreType.DMA,
            pltpu.SemaphoreType.DMA,
            pltpu.SemaphoreType.DMA,
        ],
    )
    def sc_edge(x_hbm, src_hbm, dst_hbm, e_hbm, out_hbm,
                agg, srcb0, srcb1, dstb0, dstb1, xgb0, xgb1, eb0, eb1,
                gsem0, gsem1, esem0, esem1):
        cid = lax.axis_index("c")
        sid = lax.axis_index("s")
        wid = sid * NC + cid
        srcb = (srcb0, srcb1)
        dstb = (dstb0, dstb1)
        xgb = (xgb0, xgb1)
        eb = (eb0, eb1)
        gsem = (gsem0, gsem1)
        esem = (esem0, esem1)

        def zrow(r, carry):
            for c in range(NV):
                xgb0[r, pl.ds(c * 16, 16)] = jnp.zeros((16,), jnp.float32)
            return carry

        lax.fori_loop(0, ZR, zrow, 0)
        for j in range(NZ):
            ch = sid + j * NS

            @pl.when(ch < NZCH)
            def _z():
                pltpu.sync_copy(xgb0, agg.at[pl.ds(ch * ZR, ZR)])

        plsc.subcore_barrier()

        ebase = wid * EPW

        def issue(ii, b):
            base = ebase + ii * CH
            pltpu.sync_copy(src_hbm.at[pl.ds(base, CH)], srcb[b])
            pltpu.sync_copy(dst_hbm.at[pl.ds(base, CH)], dstb[b])
            pltpu.async_copy(e_hbm.at[pl.ds(base, CH)], eb[b], esem[b])
            pltpu.async_copy(x_hbm.at[srcb[b]], xgb[b], gsem[b])

        for b in range(2):
            if b < NCHUNK:
                issue(b, b)

        def body(i2, carry):
            for b in range(2):
                ii = i2 * 2 + b

                @pl.when(ii < NCHUNK)
                def _chunk():
                    base = ebase + ii * CH
                    pltpu.make_async_copy(
                        e_hbm.at[pl.ds(base, CH)], eb[b], esem[b]).wait()
                    pltpu.make_async_copy(
                        x_hbm.at[srcb[b]], xgb[b], gsem[b]).wait()

                    def crow(r, c2):
                        for c in range(NV):
                            sl = pl.ds(c * 16, 16)
                            xgb[b][r, sl] = jnp.maximum(
                                xgb[b][r, sl] + eb[b][r, sl], 0.0)
                        return c2

                    lax.fori_loop(0, CH, crow, 0)
                    pltpu.sync_copy(xgb[b], agg.at[dstb[b]], add=True)

                    @pl.when(ii + 2 < NCHUNK)
                    def _next():
                        issue(ii + 2, b)

            return carry

        lax.fori_loop(0, (NCHUNK + 1) // 2, body, 0)

        plsc.subcore_barrier()
        for j in range(NZ):
            ch = sid + j * NS

            @pl.when(ch < NZCH)
            def _o():
                sl = pl.ds(ch * ZR, ZR)
                pltpu.sync_copy(agg.at[sl], out_hbm.at[cid, sl])

    return sc_edge


@functools.lru_cache(maxsize=None)
def _dense1(N, D, B, BN):
    G = N // BN
    SW = 3 * D

    def kern(x_ref, agg_ref, bt_ref, w1_ref, b1_ref, w2_ref, b2_ref,
             h2_ref, st_ref):
        h = x_ref[...] + agg_ref[0] + agg_ref[1]
        t = jnp.dot(h, w1_ref[...], preferred_element_type=jnp.float32) + b1_ref[...]
        t = t * jax.nn.sigmoid(t)
        h2 = jnp.dot(t, w2_ref[...], preferred_element_type=jnp.float32) + b2_ref[...]
        h2_ref[...] = h2
        bt = bt_ref[0, 0, :]
        oh = (lax.broadcasted_iota(jnp.int32, (B, BN), 0) == bt[None, :]).astype(
            jnp.float32)
        cat = jnp.concatenate([h2, h2 * h2, jnp.ones((BN, D), jnp.float32)], axis=1)

        @pl.when(pl.program_id(0) == 0)
        def _init():
            st_ref[...] = jnp.zeros_like(st_ref)

        st_ref[...] += jnp.dot(oh, cat, preferred_element_type=jnp.float32)

    return pl.pallas_call(
        kern,
        grid=(G,),
        in_specs=[
            pl.BlockSpec((BN, D), lambda i: (i, 0)),
            pl.BlockSpec((2, BN, D), lambda i: (0, i, 0)),
            pl.BlockSpec((1, 1, BN), lambda i: (i, 0, 0)),
            pl.BlockSpec((D, D), lambda i: (0, 0)),
            pl.BlockSpec((1, D), lambda i: (0, 0)),
            pl.BlockSpec((D, D), lambda i: (0, 0)),
            pl.BlockSpec((1, D), lambda i: (0, 0)),
        ],
        out_specs=[
            pl.BlockSpec((BN, D), lambda i: (i, 0)),
            pl.BlockSpec((B, SW), lambda i: (0, 0)),
        ],
        out_shape=[
            jax.ShapeDtypeStruct((N, D), jnp.float32),
            jax.ShapeDtypeStruct((B, SW), jnp.float32),
        ],
    )


@functools.lru_cache(maxsize=None)
def _dense2(N, D, B, TD, BN):
    G = N // BN
    SW = 3 * D

    def kern(x_ref, h2_ref, bt_ref, st_ref, te_ref, gw_ref, gb_ref,
             bw_ref, bb_ref, gnw_ref, gnb_ref, gms_ref, o_ref, ac_ref):
        @pl.when(pl.program_id(0) == 0)
        def _coef():
            counts = jnp.maximum(st_ref[:, 2 * D:3 * D][:, 0:1], 1.0)
            m = st_ref[:, 0:D] / counts
            eh2 = st_ref[:, D:2 * D] / counts
            ms = gms_ref[...]
            var = eh2 - m * m * ms * (2.0 - ms)
            rstd = lax.rsqrt(var + 1e-5)
            gamma = (jnp.dot(te_ref[...], gw_ref[...],
                             preferred_element_type=jnp.float32)
                     + gb_ref[...] + 1.0)
            beta = (jnp.dot(te_ref[...], bw_ref[...],
                            preferred_element_type=jnp.float32)
                    + bb_ref[...])
            w = gnw_ref[...]
            a = gamma * rstd * w
            cc = gamma * (gnb_ref[...] - m * ms * rstd * w) + beta
            ac_ref[...] = jnp.concatenate([a, cc], axis=1)

        bt = bt_ref[0, 0, :]
        oh = (lax.broadcasted_iota(jnp.int32, (BN, B), 1) == bt[:, None]).astype(
            jnp.float32)
        acg = jnp.dot(oh, ac_ref[...], preferred_element_type=jnp.float32)
        z = acg[:, 0:D] * h2_ref[...] + acg[:, D:2 * D]
        o_ref[...] = x_ref[...] + z * jax.nn.sigmoid(z)

    return pl.pallas_call(
        kern,
        grid=(G,),
        in_specs=[
            pl.BlockSpec((BN, D), lambda i: (i, 0)),
            pl.BlockSpec((BN, D), lambda i: (i, 0)),
            pl.BlockSpec((1, 1, BN), lambda i: (i, 0, 0)),
            pl.BlockSpec((B, SW), lambda i: (0, 0)),
            pl.BlockSpec((B, TD), lambda i: (0, 0)),
            pl.BlockSpec((TD, D), lambda i: (0, 0)),
            pl.BlockSpec((1, D), lambda i: (0, 0)),
            pl.BlockSpec((TD, D), lambda i: (0, 0)),
            pl.BlockSpec((1, D), lambda i: (0, 0)),
            pl.BlockSpec((1, D), lambda i: (0, 0)),
            pl.BlockSpec((1, D), lambda i: (0, 0)),
            pl.BlockSpec((1, D), lambda i: (0, 0)),
        ],
        out_specs=pl.BlockSpec((BN, D), lambda i: (i, 0)),
        out_shape=jax.ShapeDtypeStruct((N, D), jnp.float32),
        scratch_shapes=[pltpu.VMEM((B, 2 * D), jnp.float32)],
    )


def kernel(x, edge_index, edge_attr, batch, target_embeddings,
           lin_edge_W, lin_edge_b, nn_W1, nn_b1, nn_W2, nn_b2,
           gn_weight, gn_bias, gn_mean_scale,
           film_gamma_W, film_gamma_b, film_beta_W, film_beta_b):
    N, D = x.shape
    E = edge_index.shape[1]
    DE = edge_attr.shape[1]
    B, TD = target_embeddings.shape
    F = 128 // DE

    src = edge_index[0]
    dst = edge_index[1]

    w_big = jnp.kron(jnp.eye(F, dtype=jnp.float32), lin_edge_W)
    b_big = jnp.tile(lin_edge_b, F).reshape(1, F * D)
    ea2 = edge_attr.reshape(E // F, F * DE)
    e = _edge_proj(E, DE, D, 400)(ea2, w_big, b_big).reshape(E, D)

    agg_p = _sc_edge_agg(N, D, E)(x, src, dst, e)

    BN = 1000
    batch3 = batch.reshape(N // BN, 1, BN)
    h2, stats = _dense1(N, D, B, BN)(
        x, agg_p, batch3, nn_W1, nn_b1.reshape(1, D), nn_W2, nn_b2.reshape(1, D))
    out = _dense2(N, D, B, TD, BN)(
        x, h2, batch3, stats, target_embeddings,
        film_gamma_W, film_gamma_b.reshape(1, D),
        film_beta_W, film_beta_b.reshape(1, D),
        gn_weight.reshape(1, D), gn_bias.reshape(1, D), gn_mean_scale.reshape(1, D))
    return out


# speedup vs baseline: 3.3525x; 1.1156x over previous
import functools

import jax
import jax.numpy as jnp
from jax import lax
from jax.experimental import pallas as pl
from jax.experimental.pallas import tpu as pltpu
from jax.experimental.pallas import tpu_sc as plsc


def _pack_bf16_words(a, b):
    ua = lax.bitcast_convert_type(a, jnp.uint32)
    ub = lax.bitcast_convert_type(b, jnp.uint32)
    ra = (ua + jnp.uint32(0x7FFF) + ((ua >> 16) & 1)) >> 16
    rb = (ub + jnp.uint32(0x7FFF) + ((ub >> 16) & 1)) & jnp.uint32(0xFFFF0000)
    return ra | rb


@functools.lru_cache(maxsize=None)
def _edge_proj(E, DE, D, BE):
    F = 128 // DE
    R = E // F
    G = R // BE
    H = F * D // 2
    assert R % BE == 0

    def kern(ea_ref, wa_ref, wb_ref, ba_ref, bb_ref, o_ref):
        a = jnp.dot(ea_ref[...], wa_ref[...],
                    preferred_element_type=jnp.float32) + ba_ref[...]
        b = jnp.dot(ea_ref[...], wb_ref[...],
                    preferred_element_type=jnp.float32) + bb_ref[...]
        o_ref[...] = _pack_bf16_words(a, b)

    return pl.pallas_call(
        kern,
        grid=(G,),
        in_specs=[
            pl.BlockSpec((BE, F * DE), lambda i: (i, 0)),
            pl.BlockSpec((F * DE, H), lambda i: (0, 0)),
            pl.BlockSpec((F * DE, H), lambda i: (0, 0)),
            pl.BlockSpec((1, H), lambda i: (0, 0)),
            pl.BlockSpec((1, H), lambda i: (0, 0)),
        ],
        out_specs=pl.BlockSpec((BE, H), lambda i: (i, 0)),
        out_shape=jax.ShapeDtypeStruct((R, H), jnp.uint32),
    )


@functools.lru_cache(maxsize=None)
def _sc_edge_agg(N, D, E):
    info = plsc.get_sparse_core_info()
    NC, NS = info.num_cores, info.num_subcores
    NW = NC * NS
    EPW = E // NW
    CH = 80
    NCHUNK = EPW // CH
    assert EPW * NW == E and NCHUNK * CH == EPW
    ZR = CH
    NZCH = N // ZR
    assert NZCH * ZR == N
    NZ = -(-NZCH // NS)
    NV = D // 16

    mesh = plsc.VectorSubcoreMesh(core_axis_name="c", subcore_axis_name="s")

    @functools.partial(
        pl.kernel,
        mesh=mesh,
        compiler_params=pltpu.CompilerParams(needs_layout_passes=False),
        out_type=jax.ShapeDtypeStruct((NC, N, D), jnp.float32),
        scratch_types=[
            pltpu.VMEM_SHARED((N, D), jnp.float32),
            pltpu.VMEM((CH,), jnp.int32),
            pltpu.VMEM((CH,), jnp.int32),
            pltpu.VMEM((CH,), jnp.int32),
            pltpu.VMEM((CH,), jnp.int32),
            pltpu.VMEM((CH, D), jnp.float32),
            pltpu.VMEM((CH, D), jnp.float32),
            pltpu.VMEM((CH, D // 2), jnp.uint32),
            pltpu.VMEM((CH, D // 2), jnp.uint32),
            pltpu.SemaphoreType.DMA,
            pltpu.SemaphoreType.DMA,
            pltpu.SemaphoreType.DMA,
            pltpu.SemaphoreType.DMA,
        ],
    )
    def sc_edge(x_hbm, src_hbm, dst_hbm, e_hbm, out_hbm,
                agg, srcb0, srcb1, dstb0, dstb1, xgb0, xgb1, eb0, eb1,
                gsem0, gsem1, esem0, esem1):
        cid = lax.axis_index("c")
        sid = lax.axis_index("s")
        wid = sid * NC + cid
        srcb = (srcb0, srcb1)
        dstb = (dstb0, dstb1)
        xgb = (xgb0, xgb1)
        eb = (eb0, eb1)
        gsem = (gsem0, gsem1)
        esem = (esem0, esem1)

        def zrow(r, carry):
            for c in range(NV):
                xgb0[r, pl.ds(c * 16, 16)] = jnp.zeros((16,), jnp.float32)
            return carry

        lax.fori_loop(0, ZR, zrow, 0)
        for j in range(NZ):
            ch = sid + j * NS

            @pl.when(ch < NZCH)
            def _z():
                pltpu.sync_copy(xgb0, agg.at[pl.ds(ch * ZR, ZR)])

        plsc.subcore_barrier()

        ebase = wid * EPW

        def issue(ii, b):
            base = ebase + ii * CH
            pltpu.sync_copy(src_hbm.at[pl.ds(base, CH)], srcb[b])
            pltpu.sync_copy(dst_hbm.at[pl.ds(base, CH)], dstb[b])
            pltpu.async_copy(e_hbm.at[pl.ds(base, CH)], eb[b], esem[b])
            pltpu.async_copy(x_hbm.at[srcb[b]], xgb[b], gsem[b])

        for b in range(2):
            if b < NCHUNK:
                issue(b, b)

        def body(i2, carry):
            for b in range(2):
                ii = i2 * 2 + b

                @pl.when(ii < NCHUNK)
                def _chunk():
                    base = ebase + ii * CH
                    pltpu.make_async_copy(
                        e_hbm.at[pl.ds(base, CH)], eb[b], esem[b]).wait()
                    pltpu.make_async_copy(
                        x_hbm.at[srcb[b]], xgb[b], gsem[b]).wait()

                    def crow(r, c2):
                        for t in range(NV // 2):
                            ew = plsc.bitcast(
                                eb[b][r, pl.ds(t * 16, 16)], jnp.bfloat16)
                            ea_, eb_ = plsc.unpack(
                                ew, format=plsc.PackFormat.INTERLEAVED)
                            sla = pl.ds(t * 32, 16)
                            slb = pl.ds(t * 32 + 16, 16)
                            xgb[b][r, sla] = jnp.maximum(
                                xgb[b][r, sla] + ea_, 0.0)
                            xgb[b][r, slb] = jnp.maximum(
                                xgb[b][r, slb] + eb_, 0.0)
                        return c2

                    lax.fori_loop(0, CH, crow, 0)
                    pltpu.sync_copy(xgb[b], agg.at[dstb[b]], add=True)

                    @pl.when(ii + 2 < NCHUNK)
                    def _next():
                        issue(ii + 2, b)

            return carry

        lax.fori_loop(0, (NCHUNK + 1) // 2, body, 0)

        plsc.subcore_barrier()
        for j in range(NZ):
            ch = sid + j * NS

            @pl.when(ch < NZCH)
            def _o():
                sl = pl.ds(ch * ZR, ZR)
                pltpu.sync_copy(agg.at[sl], out_hbm.at[cid, sl])

    return sc_edge


@functools.lru_cache(maxsize=None)
def _dense1(N, D, B, BN):
    G = N // BN
    SW = 3 * D

    def kern(x_ref, agg_ref, bt_ref, w1_ref, b1_ref, w2_ref, b2_ref,
             h2_ref, st_ref):
        h = x_ref[...] + agg_ref[0] + agg_ref[1]
        t = jnp.dot(h, w1_ref[...], preferred_element_type=jnp.float32) + b1_ref[...]
        t = t * jax.nn.sigmoid(t)
        h2 = jnp.dot(t, w2_ref[...], preferred_element_type=jnp.float32) + b2_ref[...]
        h2_ref[...] = h2
        bt = bt_ref[0, 0, :]
        oh = (lax.broadcasted_iota(jnp.int32, (B, BN), 0) == bt[None, :]).astype(
            jnp.float32)
        cat = jnp.concatenate([h2, h2 * h2, jnp.ones((BN, D), jnp.float32)], axis=1)

        @pl.when(pl.program_id(0) == 0)
        def _init():
            st_ref[...] = jnp.zeros_like(st_ref)

        st_ref[...] += jnp.dot(oh, cat, preferred_element_type=jnp.float32)

    return pl.pallas_call(
        kern,
        grid=(G,),
        in_specs=[
            pl.BlockSpec((BN, D), lambda i: (i, 0)),
            pl.BlockSpec((2, BN, D), lambda i: (0, i, 0)),
            pl.BlockSpec((1, 1, BN), lambda i: (i, 0, 0)),
            pl.BlockSpec((D, D), lambda i: (0, 0)),
            pl.BlockSpec((1, D), lambda i: (0, 0)),
            pl.BlockSpec((D, D), lambda i: (0, 0)),
            pl.BlockSpec((1, D), lambda i: (0, 0)),
        ],
        out_specs=[
            pl.BlockSpec((BN, D), lambda i: (i, 0)),
            pl.BlockSpec((B, SW), lambda i: (0, 0)),
        ],
        out_shape=[
            jax.ShapeDtypeStruct((N, D), jnp.float32),
            jax.ShapeDtypeStruct((B, SW), jnp.float32),
        ],
    )


@functools.lru_cache(maxsize=None)
def _dense2(N, D, B, TD, BN):
    G = N // BN
    SW = 3 * D

    def kern(x_ref, h2_ref, bt_ref, st_ref, te_ref, gw_ref, gb_ref,
             bw_ref, bb_ref, gnw_ref, gnb_ref, gms_ref, o_ref, ac_ref):
        @pl.when(pl.program_id(0) == 0)
        def _coef():
            counts = jnp.maximum(st_ref[:, 2 * D:3 * D][:, 0:1], 1.0)
            m = st_ref[:, 0:D] / counts
            eh2 = st_ref[:, D:2 * D] / counts
            ms = gms_ref[...]
            var = eh2 - m * m * ms * (2.0 - ms)
            rstd = lax.rsqrt(var + 1e-5)
            gamma = (jnp.dot(te_ref[...], gw_ref[...],
                             preferred_element_type=jnp.float32)
                     + gb_ref[...] + 1.0)
            beta = (jnp.dot(te_ref[...], bw_ref[...],
                            preferred_element_type=jnp.float32)
                    + bb_ref[...])
            w = gnw_ref[...]
            a = gamma * rstd * w
            cc = gamma * (gnb_ref[...] - m * ms * rstd * w) + beta
            ac_ref[...] = jnp.concatenate([a, cc], axis=1)

        bt = bt_ref[0, 0, :]
        oh = (lax.broadcasted_iota(jnp.int32, (BN, B), 1) == bt[:, None]).astype(
            jnp.float32)
        acg = jnp.dot(oh, ac_ref[...], preferred_element_type=jnp.float32)
        z = acg[:, 0:D] * h2_ref[...] + acg[:, D:2 * D]
        o_ref[...] = x_ref[...] + z * jax.nn.sigmoid(z)

    return pl.pallas_call(
        kern,
        grid=(G,),
        in_specs=[
            pl.BlockSpec((BN, D), lambda i: (i, 0)),
            pl.BlockSpec((BN, D), lambda i: (i, 0)),
            pl.BlockSpec((1, 1, BN), lambda i: (i, 0, 0)),
            pl.BlockSpec((B, SW), lambda i: (0, 0)),
            pl.BlockSpec((B, TD), lambda i: (0, 0)),
            pl.BlockSpec((TD, D), lambda i: (0, 0)),
            pl.BlockSpec((1, D), lambda i: (0, 0)),
            pl.BlockSpec((TD, D), lambda i: (0, 0)),
            pl.BlockSpec((1, D), lambda i: (0, 0)),
            pl.BlockSpec((1, D), lambda i: (0, 0)),
            pl.BlockSpec((1, D), lambda i: (0, 0)),
            pl.BlockSpec((1, D), lambda i: (0, 0)),
        ],
        out_specs=pl.BlockSpec((BN, D), lambda i: (i, 0)),
        out_shape=jax.ShapeDtypeStruct((N, D), jnp.float32),
        scratch_shapes=[pltpu.VMEM((B, 2 * D), jnp.float32)],
    )


def kernel(x, edge_index, edge_attr, batch, target_embeddings,
           lin_edge_W, lin_edge_b, nn_W1, nn_b1, nn_W2, nn_b2,
           gn_weight, gn_bias, gn_mean_scale,
           film_gamma_W, film_gamma_b, film_beta_W, film_beta_b):
    N, D = x.shape
    E = edge_index.shape[1]
    DE = edge_attr.shape[1]
    B, TD = target_embeddings.shape
    F = 128 // DE

    src = edge_index[0]
    dst = edge_index[1]

    qa = (jnp.arange(D // 2) // 16) * 32 + (jnp.arange(D // 2) % 16)
    qb = qa + 16

    wa_big = jnp.kron(jnp.eye(F, dtype=jnp.float32), lin_edge_W[:, qa])
    wb_big = jnp.kron(jnp.eye(F, dtype=jnp.float32), lin_edge_W[:, qb])
    ba_big = jnp.tile(lin_edge_b[qa], F).reshape(1, F * D // 2)
    bb_big = jnp.tile(lin_edge_b[qb], F).reshape(1, F * D // 2)
    ea2 = edge_attr.reshape(E // F, F * DE)
    e16 = _edge_proj(E, DE, D, 400)(
        ea2, wa_big, wb_big, ba_big, bb_big).reshape(E, D // 2)

    agg_p = _sc_edge_agg(N, D, E)(x, src, dst, e16)

    BN = 1000
    batch3 = batch.reshape(N // BN, 1, BN)
    h2, stats = _dense1(N, D, B, BN)(
        x, agg_p, batch3, nn_W1, nn_b1.reshape(1, D), nn_W2, nn_b2.reshape(1, D))
    out = _dense2(N, D, B, TD, BN)(
        x, h2, batch3, stats, target_embeddings,
        film_gamma_W, film_gamma_b.reshape(1, D),
        film_beta_W, film_beta_b.reshape(1, D),
        gn_weight.reshape(1, D), gn_bias.reshape(1, D), gn_mean_scale.reshape(1, D))
    return out
